# baseline (speedup 1.0000x reference)
import functools

import jax
import jax.numpy as jnp
import numpy as np
from jax import lax
from jax.experimental import pallas as pl
from jax.experimental.pallas import tpu as pltpu
from jax.experimental.pallas import tpu_sc as plsc

B = 4096
NUM_DENSE = 13
NCAT = 26
VOCAB = 100000
D = 32
NFEAT = NCAT + 1


_VCHUNK = 2048
_NCH_T = 49
_PPC = _VCHUNK // 4
_PROWS_T = _NCH_T * _PPC
_PROWS = NCAT * _PROWS_T


def _pack_body(in_ref, out_ref):
    x = in_ref[0]
    r = lax.broadcasted_iota(jnp.int32, (D, D), 0)
    eye = (r == r.T).astype(jnp.float32)
    y = lax.dot_general(x, eye, (((0,), (0,)), ((), ())),
                        preferred_element_type=jnp.float32)
    pieces = [y[q * _PPC:(q + 1) * _PPC] for q in range(4)]
    out_ref[...] = jnp.concatenate(pieces, axis=1)


def _pack_table(tab_t):
    return pl.pallas_call(
        _pack_body,
        grid=(NCAT, _NCH_T),
        in_specs=[pl.BlockSpec((1, D, _VCHUNK), lambda t, c: (t, 0, c))],
        out_specs=pl.BlockSpec((_PPC, 4 * D), lambda t, c: (t * _NCH_T + c, 0)),
        out_shape=jax.ShapeDtypeStruct((_PROWS, 4 * D), jnp.float32),
        compiler_params=pltpu.CompilerParams(
            dimension_semantics=("arbitrary", "arbitrary")),
    )(tab_t)


_NW = 32
_ROWS = B * NCAT
_RPW = _ROWS // _NW
_RPW_PAD = 4096
_GCH = 416
_NGCH = _RPW // _GCH
_QSHIFT = 24


def _sc_gather_body(tab_hbm, idx_hbm, out_hbm,
                    idx_v, prow_v, qoff_v, stage, outb, sem):
    c = lax.axis_index("c")
    s = lax.axis_index("s")
    wid = s * 2 + c
    pltpu.sync_copy(idx_hbm.at[pl.ds(wid * _RPW_PAD, _RPW)], idx_v)

    def decode(i, carry):
        v = idx_v[pl.ds(i * 16, 16)]
        prow_v[pl.ds(i * 16, 16)] = v & jnp.int32((1 << _QSHIFT) - 1)
        qoff_v[pl.ds(i * 16, 16)] = (v >> _QSHIFT) * D
        return carry

    lax.fori_loop(0, _RPW // 16, decode, 0)

    iota = lax.iota(jnp.int32, 16)
    for k in range(_NGCH):
        pltpu.async_copy(
            tab_hbm.at[prow_v.at[pl.ds(k * _GCH, _GCH)]], stage, sem,
        ).wait()

        def extract(g, carry):
            rows = iota + g * 16
            qv = qoff_v[pl.ds(k * _GCH + g * 16, 16)]
            for j in range(D):
                vals = plsc.load_gather(stage, [rows, qv + j])
                plsc.store_scatter(outb, [rows, jnp.full((16,), j, jnp.int32)], vals)
            return carry

        lax.fori_loop(0, _GCH // 16, extract, 0)
        pltpu.sync_copy(outb, out_hbm.at[pl.ds(wid * _RPW + k * _GCH, _GCH)])


@functools.cache
def _sc_gather():
    return pl.kernel(
        _sc_gather_body,
        out_type=jax.ShapeDtypeStruct((_ROWS, D), jnp.float32),
        mesh=plsc.VectorSubcoreMesh(core_axis_name="c", subcore_axis_name="s"),
        scratch_types=[
            pltpu.VMEM((_RPW,), jnp.int32),
            pltpu.VMEM((_RPW,), jnp.int32),
            pltpu.VMEM((_RPW,), jnp.int32),
            pltpu.VMEM((_GCH, 4 * D), jnp.float32),
            pltpu.VMEM((_GCH, D), jnp.float32),
            pltpu.SemaphoreType.DMA,
        ],
        compiler_params=pltpu.CompilerParams(
            use_tc_tiling_on_sc=False, needs_layout_passes=False),
    )


_BB = 512
_GRID = B // _BB

_PERM = np.empty((D + NFEAT * NCAT // 2,), dtype=np.int32)
_PERM[:D] = np.arange(D)
_m = 0
for _s in range(1, NFEAT):
    for _j in range(NFEAT - _s):
        _i = _j + _s
        _PERM[D + _m] = D + (_i * (_i - 1)) // 2 + _j
        _m += 1


def _tc_dense_body(xt_ref, emb_ref,
                   bw0t, bb0, bw1t, bb1, bw2t, bb2,
                   tw0tp, tb0, tw1t, tb1, tw2t, tb2, tw3t, tb3, tw4t, tb4,
                   out_ref):
    f32 = jnp.float32
    h = jnp.maximum(jnp.dot(bw0t[...], xt_ref[...], preferred_element_type=f32) + bb0[...], 0.0)
    h = jnp.maximum(jnp.dot(bw1t[...], h, preferred_element_type=f32) + bb1[...], 0.0)
    bot = jnp.maximum(jnp.dot(bw2t[...], h, preferred_element_type=f32) + bb2[...], 0.0)
    embt = emb_ref[...].T
    tt = jnp.concatenate([bot, embt], axis=0)
    zs = []
    for s in range(1, NFEAT):
        w = NFEAT - s
        a = tt[: w * D, :]
        b = tt[s * D: (s + w) * D, :]
        p = (a * b).reshape(w, D, _BB)
        zs.append(jnp.sum(p, axis=1))
    rt = jnp.concatenate([bot] + zs, axis=0)
    y = jnp.maximum(jnp.dot(tw0tp[...], rt, preferred_element_type=f32) + tb0[...], 0.0)
    y = jnp.maximum(jnp.dot(tw1t[...], y, preferred_element_type=f32) + tb1[...], 0.0)
    y = jnp.maximum(jnp.dot(tw2t[...], y, preferred_element_type=f32) + tb2[...], 0.0)
    y = jnp.maximum(jnp.dot(tw3t[...], y, preferred_element_type=f32) + tb3[...], 0.0)
    y = jnp.dot(tw4t[...], y, preferred_element_type=f32) + tb4[...]
    out_ref[...] = y.reshape(1, 1, _BB)


def _const_spec(shape):
    return pl.BlockSpec(shape, lambda i: tuple(0 for _ in shape))


def _tc_dense(xt, emb2, weights):
    in_specs = [
        pl.BlockSpec((NUM_DENSE, _BB), lambda i: (0, i)),
        pl.BlockSpec((_BB, NCAT * D), lambda i: (i, 0)),
    ] + [_const_spec(w.shape) for w in weights]
    return pl.pallas_call(
        _tc_dense_body,
        grid=(_GRID,),
        in_specs=in_specs,
        out_specs=pl.BlockSpec((1, 1, _BB), lambda i: (i, 0, 0)),
        out_shape=jax.ShapeDtypeStruct((_GRID, 1, _BB), jnp.float32),
        compiler_params=pltpu.CompilerParams(
            dimension_semantics=("arbitrary",)),
    )(xt, emb2, *weights)


def kernel(numerical_input, categorical_input, tables,
           bw0, bb0, bw1, bb1, bw2, bb2,
           tw0, tb0, tw1, tb1, tw2, tb2, tw3, tb3, tw4, tb4):
    v = categorical_input
    toff = jnp.arange(NCAT, dtype=jnp.int32) * _PROWS_T
    prow = toff + (v >> 11) * _PPC + (v & (_PPC - 1))
    idxp = prow | (((v >> 9) & 3) << _QSHIFT)
    idx3 = idxp.reshape(_NW, _RPW)
    idx2 = jnp.pad(idx3, ((0, 0), (0, _RPW_PAD - _RPW))).reshape(_NW * _RPW_PAD)

    tab_packed = _pack_table(jnp.transpose(tables, (0, 2, 1)))
    emb = _sc_gather()(tab_packed, idx2)
    emb2 = emb.reshape(B, NCAT * D)

    weights = (
        bw0.T, bb0.reshape(-1, 1), bw1.T, bb1.reshape(-1, 1),
        bw2.T, bb2.reshape(-1, 1),
        tw0.T[:, _PERM], tb0.reshape(-1, 1), tw1.T, tb1.reshape(-1, 1),
        tw2.T, tb2.reshape(-1, 1), tw3.T, tb3.reshape(-1, 1),
        tw4.T, tb4.reshape(-1, 1),
    )
    out = _tc_dense(numerical_input.T, emb2, weights)
    return out.reshape(B)


# speedup vs baseline: 2.1078x; 2.1078x over previous
import functools

import jax
import jax.numpy as jnp
import numpy as np
from jax import lax
from jax.experimental import pallas as pl
from jax.experimental.pallas import tpu as pltpu
from jax.experimental.pallas import tpu_sc as plsc

B = 4096
NUM_DENSE = 13
NCAT = 26
VOCAB = 100000
D = 32
NFEAT = NCAT + 1


_VCHUNK = 2048
_NCH_T = 49
_VPAD = _NCH_T * _VCHUNK
_TG = (NCAT + 3) // 4
_PROWS = _TG * _VPAD


def _pack_body(in_ref, eye_ref, out_ref):
    x = in_ref[...].reshape(4 * D, _VCHUNK)
    g = pl.program_id(0)
    nvalid = jnp.where(g == _TG - 1, (NCAT - 4 * (_TG - 1)) * D, 4 * D)
    row = lax.broadcasted_iota(jnp.int32, (4 * D, _VCHUNK), 0)
    x = jnp.where(row < nvalid, x, 0.0)
    out_ref[...] = lax.dot_general(
        x, eye_ref[...], (((0,), (0,)), ((), ())),
        preferred_element_type=jnp.float32)


def _pack_table(tab_t, eye):
    return pl.pallas_call(
        _pack_body,
        grid=(_TG, _NCH_T),
        in_specs=[
            pl.BlockSpec((4, D, _VCHUNK), lambda g, c: (g, 0, c)),
            pl.BlockSpec((4 * D, 4 * D), lambda g, c: (0, 0)),
        ],
        out_specs=pl.BlockSpec((_VCHUNK, 4 * D), lambda g, c: (g * _NCH_T + c, 0)),
        out_shape=jax.ShapeDtypeStruct((_PROWS, 4 * D), jnp.float32),
        compiler_params=pltpu.CompilerParams(
            dimension_semantics=("arbitrary", "arbitrary")),
    )(tab_t, eye)


_NW = 32
_ROWS = B * NCAT
_RPW = _ROWS // _NW
_RPW_PAD = 4096
_GCH = 416
_NGCH = _RPW // _GCH
_QSHIFT = 24


def _sc_gather_body(tab_hbm, idx_hbm, out_hbm,
                    idx_v, prow_v, qoff_v, stage, outb, sem):
    c = lax.axis_index("c")
    s = lax.axis_index("s")
    wid = s * 2 + c
    pltpu.sync_copy(idx_hbm.at[pl.ds(wid * _RPW_PAD, _RPW)], idx_v)

    def decode(i, carry):
        v = idx_v[pl.ds(i * 16, 16)]
        prow_v[pl.ds(i * 16, 16)] = v & jnp.int32((1 << _QSHIFT) - 1)
        qoff_v[pl.ds(i * 16, 16)] = (v >> _QSHIFT) * D
        return carry

    lax.fori_loop(0, _RPW // 16, decode, 0)

    iota = lax.iota(jnp.int32, 16)
    for k in range(_NGCH):
        pltpu.async_copy(
            tab_hbm.at[prow_v.at[pl.ds(k * _GCH, _GCH)]], stage, sem,
        ).wait()

        def extract(g, carry):
            rows = iota + g * 16
            qv = qoff_v[pl.ds(k * _GCH + g * 16, 16)]
            for j in range(D):
                vals = plsc.load_gather(stage, [rows, qv + j])
                plsc.store_scatter(outb, [rows, jnp.full((16,), j, jnp.int32)], vals)
            return carry

        lax.fori_loop(0, _GCH // 16, extract, 0)
        pltpu.sync_copy(outb, out_hbm.at[pl.ds(wid * _RPW + k * _GCH, _GCH)])


@functools.cache
def _sc_gather():
    return pl.kernel(
        _sc_gather_body,
        out_type=jax.ShapeDtypeStruct((_ROWS, D), jnp.float32),
        mesh=plsc.VectorSubcoreMesh(core_axis_name="c", subcore_axis_name="s"),
        scratch_types=[
            pltpu.VMEM((_RPW,), jnp.int32),
            pltpu.VMEM((_RPW,), jnp.int32),
            pltpu.VMEM((_RPW,), jnp.int32),
            pltpu.VMEM((_GCH, 4 * D), jnp.float32),
            pltpu.VMEM((_GCH, D), jnp.float32),
            pltpu.SemaphoreType.DMA,
        ],
        compiler_params=pltpu.CompilerParams(
            use_tc_tiling_on_sc=False, needs_layout_passes=False),
    )


_BB = 512
_GRID = B // _BB

_PERM = np.empty((D + NFEAT * NCAT // 2,), dtype=np.int32)
_PERM[:D] = np.arange(D)
_m = 0
for _s in range(1, NFEAT):
    for _j in range(NFEAT - _s):
        _i = _j + _s
        _PERM[D + _m] = D + (_i * (_i - 1)) // 2 + _j
        _m += 1


def _tc_dense_body(xt_ref, emb_ref,
                   bw0t, bb0, bw1t, bb1, bw2t, bb2,
                   tw0tp, tb0, tw1t, tb1, tw2t, tb2, tw3t, tb3, tw4t, tb4,
                   out_ref):
    f32 = jnp.float32
    h = jnp.maximum(jnp.dot(bw0t[...], xt_ref[...], preferred_element_type=f32) + bb0[...], 0.0)
    h = jnp.maximum(jnp.dot(bw1t[...], h, preferred_element_type=f32) + bb1[...], 0.0)
    bot = jnp.maximum(jnp.dot(bw2t[...], h, preferred_element_type=f32) + bb2[...], 0.0)
    embt = emb_ref[...].T
    tt = jnp.concatenate([bot, embt], axis=0)
    zs = []
    for s in range(1, NFEAT):
        w = NFEAT - s
        a = tt[: w * D, :]
        b = tt[s * D: (s + w) * D, :]
        p = (a * b).reshape(w, D, _BB)
        zs.append(jnp.sum(p, axis=1))
    rt = jnp.concatenate([bot] + zs, axis=0)
    y = jnp.maximum(jnp.dot(tw0tp[...], rt, preferred_element_type=f32) + tb0[...], 0.0)
    y = jnp.maximum(jnp.dot(tw1t[...], y, preferred_element_type=f32) + tb1[...], 0.0)
    y = jnp.maximum(jnp.dot(tw2t[...], y, preferred_element_type=f32) + tb2[...], 0.0)
    y = jnp.maximum(jnp.dot(tw3t[...], y, preferred_element_type=f32) + tb3[...], 0.0)
    y = jnp.dot(tw4t[...], y, preferred_element_type=f32) + tb4[...]
    out_ref[...] = y.reshape(1, 1, _BB)


def _const_spec(shape):
    return pl.BlockSpec(shape, lambda i: tuple(0 for _ in shape))


def _tc_dense(xt, emb2, weights):
    in_specs = [
        pl.BlockSpec((NUM_DENSE, _BB), lambda i: (0, i)),
        pl.BlockSpec((_BB, NCAT * D), lambda i: (i, 0)),
    ] + [_const_spec(w.shape) for w in weights]
    return pl.pallas_call(
        _tc_dense_body,
        grid=(_GRID,),
        in_specs=in_specs,
        out_specs=pl.BlockSpec((1, 1, _BB), lambda i: (i, 0, 0)),
        out_shape=jax.ShapeDtypeStruct((_GRID, 1, _BB), jnp.float32),
        compiler_params=pltpu.CompilerParams(
            dimension_semantics=("arbitrary",)),
    )(xt, emb2, *weights)


def kernel(numerical_input, categorical_input, tables,
           bw0, bb0, bw1, bb1, bw2, bb2,
           tw0, tb0, tw1, tb1, tw2, tb2, tw3, tb3, tw4, tb4):
    v = categorical_input
    t = jnp.arange(NCAT, dtype=jnp.int32)
    prow = (t // 4) * _VPAD + v
    idxp = prow | ((t % 4) << _QSHIFT)
    idx3 = idxp.reshape(_NW, _RPW)
    idx2 = jnp.pad(idx3, ((0, 0), (0, _RPW_PAD - _RPW))).reshape(_NW * _RPW_PAD)

    tab_packed = _pack_table(jnp.transpose(tables, (0, 2, 1)),
                             jnp.eye(4 * D, dtype=jnp.float32))
    emb = _sc_gather()(tab_packed, idx2)
    emb2 = emb.reshape(B, NCAT * D)

    weights = (
        bw0.T, bb0.reshape(-1, 1), bw1.T, bb1.reshape(-1, 1),
        bw2.T, bb2.reshape(-1, 1),
        tw0.T[:, _PERM], tb0.reshape(-1, 1), tw1.T, tb1.reshape(-1, 1),
        tw2.T, tb2.reshape(-1, 1), tw3.T, tb3.reshape(-1, 1),
        tw4.T, tb4.reshape(-1, 1),
    )
    out = _tc_dense(numerical_input.T, emb2, weights)
    return out.reshape(B)


# speedup vs baseline: 2.7043x; 1.2830x over previous
import functools

import jax
import jax.numpy as jnp
import numpy as np
from jax import lax
from jax.experimental import pallas as pl
from jax.experimental.pallas import tpu as pltpu
from jax.experimental.pallas import tpu_sc as plsc

B = 4096
NUM_DENSE = 13
NCAT = 26
VOCAB = 100000
D = 32
NFEAT = NCAT + 1


_VCHUNK = 2048
_NCH_T = 49
_VPAD = _NCH_T * _VCHUNK
_TG = (NCAT + 3) // 4
_PROWS = _TG * _VPAD


def _pack_body(in_ref, eye_ref, out_ref):
    x = in_ref[...].reshape(4 * D, _VCHUNK)
    g = pl.program_id(0)
    nvalid = jnp.where(g == _TG - 1, (NCAT - 4 * (_TG - 1)) * D, 4 * D)
    row = lax.broadcasted_iota(jnp.int32, (4 * D, _VCHUNK), 0)
    x = jnp.where(row < nvalid, x, 0.0)
    out_ref[...] = lax.dot_general(
        x, eye_ref[...], (((0,), (0,)), ((), ())),
        preferred_element_type=jnp.float32)


def _pack_table(tab_t, eye):
    return pl.pallas_call(
        _pack_body,
        grid=(_TG, _NCH_T),
        in_specs=[
            pl.BlockSpec((4, D, _VCHUNK), lambda g, c: (g, 0, c)),
            pl.BlockSpec((4 * D, 4 * D), lambda g, c: (0, 0)),
        ],
        out_specs=pl.BlockSpec((_VCHUNK, 4 * D), lambda g, c: (g * _NCH_T + c, 0)),
        out_shape=jax.ShapeDtypeStruct((_PROWS, 4 * D), jnp.float32),
        compiler_params=pltpu.CompilerParams(
            dimension_semantics=("arbitrary", "arbitrary")),
    )(tab_t, eye)


_NW = 32
_ROWS = B * NCAT
_RPW = _ROWS // _NW
_RPW_PAD = 4096


def _sc_gather_body(tab_hbm, idx_hbm, out_hbm, idx_v, rows_v, sem):
    c = lax.axis_index("c")
    s = lax.axis_index("s")
    wid = s * 2 + c
    pltpu.sync_copy(idx_hbm.at[pl.ds(wid * _RPW_PAD, _RPW)], idx_v)
    pltpu.async_copy(tab_hbm.at[idx_v], rows_v, sem).wait()
    pltpu.sync_copy(rows_v, out_hbm.at[pl.ds(wid * _RPW, _RPW)])


@functools.cache
def _sc_gather():
    return pl.kernel(
        _sc_gather_body,
        out_type=jax.ShapeDtypeStruct((_ROWS, D), jnp.float32),
        mesh=plsc.VectorSubcoreMesh(core_axis_name="c", subcore_axis_name="s"),
        scratch_types=[
            pltpu.VMEM((_RPW,), jnp.int32),
            pltpu.VMEM((_RPW, D), jnp.float32),
            pltpu.SemaphoreType.DMA,
        ],
        compiler_params=pltpu.CompilerParams(use_tc_tiling_on_sc=False),
    )


_BB = 512
_GRID = B // _BB

_PERM = np.empty((D + NFEAT * NCAT // 2,), dtype=np.int32)
_PERM[:D] = np.arange(D)
_m = 0
for _s in range(1, NFEAT):
    for _j in range(NFEAT - _s):
        _i = _j + _s
        _PERM[D + _m] = D + (_i * (_i - 1)) // 2 + _j
        _m += 1


def _tc_dense_body(xt_ref, emb_ref,
                   bw0t, bb0, bw1t, bb1, bw2t, bb2,
                   tw0tp, tb0, tw1t, tb1, tw2t, tb2, tw3t, tb3, tw4t, tb4,
                   out_ref):
    f32 = jnp.float32
    h = jnp.maximum(jnp.dot(bw0t[...], xt_ref[...], preferred_element_type=f32) + bb0[...], 0.0)
    h = jnp.maximum(jnp.dot(bw1t[...], h, preferred_element_type=f32) + bb1[...], 0.0)
    bot = jnp.maximum(jnp.dot(bw2t[...], h, preferred_element_type=f32) + bb2[...], 0.0)
    embt = emb_ref[...].T
    tt = jnp.concatenate([bot, embt], axis=0)
    zs = []
    for s in range(1, NFEAT):
        w = NFEAT - s
        a = tt[: w * D, :]
        b = tt[s * D: (s + w) * D, :]
        p = (a * b).reshape(w, D, _BB)
        zs.append(jnp.sum(p, axis=1))
    rt = jnp.concatenate([bot] + zs, axis=0)
    y = jnp.maximum(jnp.dot(tw0tp[...], rt, preferred_element_type=f32) + tb0[...], 0.0)
    y = jnp.maximum(jnp.dot(tw1t[...], y, preferred_element_type=f32) + tb1[...], 0.0)
    y = jnp.maximum(jnp.dot(tw2t[...], y, preferred_element_type=f32) + tb2[...], 0.0)
    y = jnp.maximum(jnp.dot(tw3t[...], y, preferred_element_type=f32) + tb3[...], 0.0)
    y = jnp.dot(tw4t[...], y, preferred_element_type=f32) + tb4[...]
    out_ref[...] = y.reshape(1, 1, _BB)


def _const_spec(shape):
    return pl.BlockSpec(shape, lambda i: tuple(0 for _ in shape))


def _tc_dense(xt, emb2, weights):
    in_specs = [
        pl.BlockSpec((NUM_DENSE, _BB), lambda i: (0, i)),
        pl.BlockSpec((_BB, NCAT * D), lambda i: (i, 0)),
    ] + [_const_spec(w.shape) for w in weights]
    return pl.pallas_call(
        _tc_dense_body,
        grid=(_GRID,),
        in_specs=in_specs,
        out_specs=pl.BlockSpec((1, 1, _BB), lambda i: (i, 0, 0)),
        out_shape=jax.ShapeDtypeStruct((_GRID, 1, _BB), jnp.float32),
        compiler_params=pltpu.CompilerParams(
            dimension_semantics=("arbitrary",)),
    )(xt, emb2, *weights)


def kernel(numerical_input, categorical_input, tables,
           bw0, bb0, bw1, bb1, bw2, bb2,
           tw0, tb0, tw1, tb1, tw2, tb2, tw3, tb3, tw4, tb4):
    v = categorical_input
    t = jnp.arange(NCAT, dtype=jnp.int32)
    idxp = ((t // 4) * _VPAD + v) * 4 + (t % 4)
    idx3 = idxp.reshape(_NW, _RPW)
    idx2 = jnp.pad(idx3, ((0, 0), (0, _RPW_PAD - _RPW))).reshape(_NW * _RPW_PAD)

    tab_packed = _pack_table(jnp.transpose(tables, (0, 2, 1)),
                             jnp.eye(4 * D, dtype=jnp.float32))
    tab4 = tab_packed.reshape(4 * _PROWS, D)
    emb = _sc_gather()(tab4, idx2)
    emb2 = emb.reshape(B, NCAT * D)

    weights = (
        bw0.T, bb0.reshape(-1, 1), bw1.T, bb1.reshape(-1, 1),
        bw2.T, bb2.reshape(-1, 1),
        tw0.T[:, _PERM], tb0.reshape(-1, 1), tw1.T, tb1.reshape(-1, 1),
        tw2.T, tb2.reshape(-1, 1), tw3.T, tb3.reshape(-1, 1),
        tw4.T, tb4.reshape(-1, 1),
    )
    out = _tc_dense(numerical_input.T, emb2, weights)
    return out.reshape(B)


# speedup vs baseline: 3.4296x; 1.2682x over previous
import functools

import jax
import jax.numpy as jnp
import numpy as np
from jax import lax
from jax.experimental import pallas as pl
from jax.experimental.pallas import tpu as pltpu
from jax.experimental.pallas import tpu_sc as plsc

B = 4096
NUM_DENSE = 13
NCAT = 26
VOCAB = 100000
D = 32
NFEAT = NCAT + 1


_VCHUNK = 4096
_NCH_T = 25
_VPAD = _NCH_T * _VCHUNK
_TG = (NCAT + 3) // 4
_PROWS = _TG * _VPAD


def _pack_body(in_ref, eye_ref, out_ref):
    x = in_ref[...].reshape(4 * D, _VCHUNK)
    g = pl.program_id(0)
    nvalid = jnp.where(g == _TG - 1, (NCAT - 4 * (_TG - 1)) * D, 4 * D)
    row = lax.broadcasted_iota(jnp.int32, (4 * D, _VCHUNK), 0)
    x = jnp.where(row < nvalid, x, 0.0)
    out_ref[...] = lax.dot_general(
        x, eye_ref[...], (((0,), (0,)), ((), ())),
        preferred_element_type=jnp.float32)


def _pack_table(tab_t, eye):
    return pl.pallas_call(
        _pack_body,
        grid=(_TG, _NCH_T),
        in_specs=[
            pl.BlockSpec((4, D, _VCHUNK), lambda g, c: (g, 0, c)),
            pl.BlockSpec((4 * D, 4 * D), lambda g, c: (0, 0)),
        ],
        out_specs=pl.BlockSpec((_VCHUNK, 4 * D), lambda g, c: (g * _NCH_T + c, 0)),
        out_shape=jax.ShapeDtypeStruct((_PROWS, 4 * D), jnp.float32),
        compiler_params=pltpu.CompilerParams(
            dimension_semantics=("arbitrary", "arbitrary")),
    )(tab_t, eye)


_NW = 32
_ROWS = B * NCAT
_RPW = _ROWS // _NW
_RPW_PAD = 4096


def _sc_gather_body(tab_hbm, idx_hbm, out_hbm, idx_v, rows_v, sem):
    c = lax.axis_index("c")
    s = lax.axis_index("s")
    wid = s * 2 + c
    pltpu.sync_copy(idx_hbm.at[pl.ds(wid * _RPW_PAD, _RPW)], idx_v)
    pltpu.async_copy(tab_hbm.at[idx_v], rows_v, sem).wait()
    pltpu.sync_copy(rows_v, out_hbm.at[pl.ds(wid * _RPW, _RPW)])


@functools.cache
def _sc_gather():
    return pl.kernel(
        _sc_gather_body,
        out_type=jax.ShapeDtypeStruct((_ROWS, D), jnp.float32),
        mesh=plsc.VectorSubcoreMesh(core_axis_name="c", subcore_axis_name="s"),
        scratch_types=[
            pltpu.VMEM((_RPW,), jnp.int32),
            pltpu.VMEM((_RPW, D), jnp.float32),
            pltpu.SemaphoreType.DMA,
        ],
        compiler_params=pltpu.CompilerParams(use_tc_tiling_on_sc=False),
    )


_BB = 512
_GRID = B // _BB

_PERM = np.empty((D + NFEAT * NCAT // 2,), dtype=np.int32)
_PERM[:D] = np.arange(D)
_m = 0
for _s in range(1, NFEAT):
    for _j in range(NFEAT - _s):
        _i = _j + _s
        _PERM[D + _m] = D + (_i * (_i - 1)) // 2 + _j
        _m += 1


def _tc_dense_body(xt_ref, emb_ref,
                   bw0t, bb0, bw1t, bb1, bw2t, bb2,
                   tw0tp, tb0, tw1t, tb1, tw2t, tb2, tw3t, tb3, tw4t, tb4,
                   out_ref):
    f32 = jnp.float32
    h = jnp.maximum(jnp.dot(bw0t[...], xt_ref[...], preferred_element_type=f32) + bb0[...], 0.0)
    h = jnp.maximum(jnp.dot(bw1t[...], h, preferred_element_type=f32) + bb1[...], 0.0)
    bot = jnp.maximum(jnp.dot(bw2t[...], h, preferred_element_type=f32) + bb2[...], 0.0)
    embt = emb_ref[...].T
    tt = jnp.concatenate([bot, embt], axis=0)
    zs = []
    for s in range(1, NFEAT):
        w = NFEAT - s
        a = tt[: w * D, :]
        b = tt[s * D: (s + w) * D, :]
        p = (a * b).reshape(w, D, _BB)
        zs.append(jnp.sum(p, axis=1))
    rt = jnp.concatenate([bot] + zs, axis=0)
    y = jnp.maximum(jnp.dot(tw0tp[...], rt, preferred_element_type=f32) + tb0[...], 0.0)
    y = jnp.maximum(jnp.dot(tw1t[...], y, preferred_element_type=f32) + tb1[...], 0.0)
    y = jnp.maximum(jnp.dot(tw2t[...], y, preferred_element_type=f32) + tb2[...], 0.0)
    y = jnp.maximum(jnp.dot(tw3t[...], y, preferred_element_type=f32) + tb3[...], 0.0)
    y = jnp.dot(tw4t[...], y, preferred_element_type=f32) + tb4[...]
    out_ref[...] = y.reshape(1, 1, _BB)


def _const_spec(shape):
    return pl.BlockSpec(shape, lambda i: tuple(0 for _ in shape))


def _tc_dense(xt, emb2, weights):
    in_specs = [
        pl.BlockSpec((NUM_DENSE, _BB), lambda i: (0, i)),
        pl.BlockSpec((_BB, NCAT * D), lambda i: (i, 0)),
    ] + [_const_spec(w.shape) for w in weights]
    return pl.pallas_call(
        _tc_dense_body,
        grid=(_GRID,),
        in_specs=in_specs,
        out_specs=pl.BlockSpec((1, 1, _BB), lambda i: (i, 0, 0)),
        out_shape=jax.ShapeDtypeStruct((_GRID, 1, _BB), jnp.float32),
        compiler_params=pltpu.CompilerParams(
            dimension_semantics=("arbitrary",)),
    )(xt, emb2, *weights)


def kernel(numerical_input, categorical_input, tables,
           bw0, bb0, bw1, bb1, bw2, bb2,
           tw0, tb0, tw1, tb1, tw2, tb2, tw3, tb3, tw4, tb4):
    v = categorical_input
    t = jnp.arange(NCAT, dtype=jnp.int32)
    idxp = ((t // 4) * _VPAD + v) * 4 + (t % 4)
    idx3 = idxp.reshape(_NW, _RPW)
    idx2 = jnp.pad(idx3, ((0, 0), (0, _RPW_PAD - _RPW))).reshape(_NW * _RPW_PAD)

    tab_packed = _pack_table(jnp.transpose(tables, (0, 2, 1)),
                             jnp.eye(4 * D, dtype=jnp.float32))
    tab4 = tab_packed.reshape(4 * _PROWS, D)
    emb = _sc_gather()(tab4, idx2)
    emb2 = emb.reshape(B, NCAT * D)

    weights = (
        bw0.T, bb0.reshape(-1, 1), bw1.T, bb1.reshape(-1, 1),
        bw2.T, bb2.reshape(-1, 1),
        tw0.T[:, _PERM], tb0.reshape(-1, 1), tw1.T, tb1.reshape(-1, 1),
        tw2.T, tb2.reshape(-1, 1), tw3.T, tb3.reshape(-1, 1),
        tw4.T, tb4.reshape(-1, 1),
    )
    out = _tc_dense(numerical_input.T, emb2, weights)
    return out.reshape(B)


# speedup vs baseline: 3.7994x; 1.1078x over previous
import functools

import jax
import jax.numpy as jnp
import numpy as np
from jax import lax
from jax.experimental import pallas as pl
from jax.experimental.pallas import tpu as pltpu
from jax.experimental.pallas import tpu_sc as plsc

B = 4096
NUM_DENSE = 13
NCAT = 26
VOCAB = 100000
D = 32
NFEAT = NCAT + 1


_VCHUNK = 8192
_NCH_T = 13
_VPAD = _NCH_T * _VCHUNK
_TG = (NCAT + 3) // 4
_PROWS = _TG * _VPAD


def _pack_body(in_ref, eye_ref, out_ref):
    x = in_ref[...].reshape(4 * D, _VCHUNK)
    g = pl.program_id(0)
    nvalid = jnp.where(g == _TG - 1, (NCAT - 4 * (_TG - 1)) * D, 4 * D)
    row = lax.broadcasted_iota(jnp.int32, (4 * D, _VCHUNK), 0)
    x = jnp.where(row < nvalid, x, 0.0)
    out_ref[...] = lax.dot_general(
        x, eye_ref[...], (((0,), (0,)), ((), ())),
        preferred_element_type=jnp.float32)


def _pack_table(tab_t, eye):
    return pl.pallas_call(
        _pack_body,
        grid=(_TG, _NCH_T),
        in_specs=[
            pl.BlockSpec((4, D, _VCHUNK), lambda g, c: (g, 0, c)),
            pl.BlockSpec((4 * D, 4 * D), lambda g, c: (0, 0)),
        ],
        out_specs=pl.BlockSpec((_VCHUNK, 4 * D), lambda g, c: (g * _NCH_T + c, 0)),
        out_shape=jax.ShapeDtypeStruct((_PROWS, 4 * D), jnp.float32),
        compiler_params=pltpu.CompilerParams(
            dimension_semantics=("arbitrary", "arbitrary")),
    )(tab_t, eye)


_NW = 32
_ROWS = B * NCAT
_RPW = _ROWS // _NW
_RPW_PAD = 4096


def _sc_gather_body(tab_hbm, idx_hbm, out_hbm, idx_v, rows_v, sem):
    c = lax.axis_index("c")
    s = lax.axis_index("s")
    wid = s * 2 + c
    pltpu.sync_copy(idx_hbm.at[pl.ds(wid * _RPW_PAD, _RPW)], idx_v)
    pltpu.async_copy(tab_hbm.at[idx_v], rows_v, sem).wait()
    pltpu.sync_copy(rows_v, out_hbm.at[pl.ds(wid * _RPW, _RPW)])


@functools.cache
def _sc_gather():
    return pl.kernel(
        _sc_gather_body,
        out_type=jax.ShapeDtypeStruct((_ROWS, D), jnp.float32),
        mesh=plsc.VectorSubcoreMesh(core_axis_name="c", subcore_axis_name="s"),
        scratch_types=[
            pltpu.VMEM((_RPW,), jnp.int32),
            pltpu.VMEM((_RPW, D), jnp.float32),
            pltpu.SemaphoreType.DMA,
        ],
        compiler_params=pltpu.CompilerParams(use_tc_tiling_on_sc=False),
    )


_BB = 512
_GRID = B // _BB

_PERM = np.empty((D + NFEAT * NCAT // 2,), dtype=np.int32)
_PERM[:D] = np.arange(D)
_m = 0
for _s in range(1, NFEAT):
    for _j in range(NFEAT - _s):
        _i = _j + _s
        _PERM[D + _m] = D + (_i * (_i - 1)) // 2 + _j
        _m += 1


def _tc_dense_body(xt_ref, emb_ref,
                   bw0t, bb0, bw1t, bb1, bw2t, bb2,
                   tw0tp, tb0, tw1t, tb1, tw2t, tb2, tw3t, tb3, tw4t, tb4,
                   out_ref):
    f32 = jnp.float32
    h = jnp.maximum(jnp.dot(bw0t[...], xt_ref[...], preferred_element_type=f32) + bb0[...], 0.0)
    h = jnp.maximum(jnp.dot(bw1t[...], h, preferred_element_type=f32) + bb1[...], 0.0)
    bot = jnp.maximum(jnp.dot(bw2t[...], h, preferred_element_type=f32) + bb2[...], 0.0)
    embt = emb_ref[...].T
    tt = jnp.concatenate([bot, embt], axis=0)
    zs = []
    for s in range(1, NFEAT):
        w = NFEAT - s
        a = tt[: w * D, :]
        b = tt[s * D: (s + w) * D, :]
        p = (a * b).reshape(w, D, _BB)
        zs.append(jnp.sum(p, axis=1))
    rt = jnp.concatenate([bot] + zs, axis=0)
    y = jnp.maximum(jnp.dot(tw0tp[...], rt, preferred_element_type=f32) + tb0[...], 0.0)
    y = jnp.maximum(jnp.dot(tw1t[...], y, preferred_element_type=f32) + tb1[...], 0.0)
    y = jnp.maximum(jnp.dot(tw2t[...], y, preferred_element_type=f32) + tb2[...], 0.0)
    y = jnp.maximum(jnp.dot(tw3t[...], y, preferred_element_type=f32) + tb3[...], 0.0)
    y = jnp.dot(tw4t[...], y, preferred_element_type=f32) + tb4[...]
    out_ref[...] = y.reshape(1, 1, _BB)


def _const_spec(shape):
    return pl.BlockSpec(shape, lambda i: tuple(0 for _ in shape))


def _tc_dense(xt, emb2, weights):
    in_specs = [
        pl.BlockSpec((NUM_DENSE, _BB), lambda i: (0, i)),
        pl.BlockSpec((_BB, NCAT * D), lambda i: (i, 0)),
    ] + [_const_spec(w.shape) for w in weights]
    return pl.pallas_call(
        _tc_dense_body,
        grid=(_GRID,),
        in_specs=in_specs,
        out_specs=pl.BlockSpec((1, 1, _BB), lambda i: (i, 0, 0)),
        out_shape=jax.ShapeDtypeStruct((_GRID, 1, _BB), jnp.float32),
        compiler_params=pltpu.CompilerParams(
            dimension_semantics=("arbitrary",)),
    )(xt, emb2, *weights)


def kernel(numerical_input, categorical_input, tables,
           bw0, bb0, bw1, bb1, bw2, bb2,
           tw0, tb0, tw1, tb1, tw2, tb2, tw3, tb3, tw4, tb4):
    v = categorical_input
    t = jnp.arange(NCAT, dtype=jnp.int32)
    idxp = ((t // 4) * _VPAD + v) * 4 + (t % 4)
    idx3 = idxp.reshape(_NW, _RPW)
    idx2 = jnp.pad(idx3, ((0, 0), (0, _RPW_PAD - _RPW))).reshape(_NW * _RPW_PAD)

    tab_packed = _pack_table(jnp.transpose(tables, (0, 2, 1)),
                             jnp.eye(4 * D, dtype=jnp.float32))
    tab4 = tab_packed.reshape(4 * _PROWS, D)
    emb = _sc_gather()(tab4, idx2)
    emb2 = emb.reshape(B, NCAT * D)

    weights = (
        bw0.T, bb0.reshape(-1, 1), bw1.T, bb1.reshape(-1, 1),
        bw2.T, bb2.reshape(-1, 1),
        tw0.T[:, _PERM], tb0.reshape(-1, 1), tw1.T, tb1.reshape(-1, 1),
        tw2.T, tb2.reshape(-1, 1), tw3.T, tb3.reshape(-1, 1),
        tw4.T, tb4.reshape(-1, 1),
    )
    out = _tc_dense(numerical_input.T, emb2, weights)
    return out.reshape(B)
